# baseline (speedup 1.0000x reference)
import functools

import jax
import jax.numpy as jnp
import numpy as np
from jax import lax
from jax.experimental import pallas as pl
from jax.experimental.pallas import tpu as pltpu
from jax.experimental.pallas import tpu_sc as plsc


def _span_mask(key, num_rows, max_row_len, span_len, max_mask_prob):
    row_lens = jnp.full((num_rows,), max_row_len, dtype=jnp.int32)
    num_spans = int(np.float32(max_mask_prob / span_len) * np.float32(max_row_len - 1))
    k1, k2 = jax.random.split(key)
    span_start_range = row_lens - span_len + 1
    span_start_range = jnp.repeat(span_start_range, num_spans)
    rand_scales = jax.random.uniform(k1, (num_rows * num_spans,), dtype=jnp.float32)
    span_offsets = (span_start_range.astype(jnp.float32) * rand_scales).astype(jnp.int32)
    span_offsets = span_offsets.reshape(num_rows, num_spans)
    span_offsets = jnp.repeat(span_offsets, span_len, axis=1)
    idx = jnp.tile(jnp.arange(span_len, dtype=jnp.int32), num_spans)[None, :]
    indices = span_offsets + idx
    row_ids = jnp.arange(num_rows, dtype=jnp.int32)[:, None]
    float_mask = jnp.zeros((num_rows, max_row_len), dtype=jnp.float32).at[row_ids, indices].set(1.0)
    min_num_masked = jnp.count_nonzero(float_mask, axis=-1).min()
    scores = jnp.where(float_mask > 0, jax.random.uniform(k2, float_mask.shape), -1.0)
    k_max = num_spans * span_len
    _, topk_idx = jax.lax.top_k(scores, k_max)
    keep = jnp.arange(k_max) < min_num_masked
    bool_mask = jnp.zeros((num_rows, max_row_len), dtype=bool).at[row_ids, topk_idx].set(keep)
    return bool_mask


_MASK_NP = np.asarray(_span_mask(jax.random.key(42), 32, 2048, 10, 0.65))

_NW = 32
_CHUNK = 80
_ECHUNK = 64


def _pad_chunks(idx, n_chunks, chunk):
    out = np.full((n_chunks * chunk,), idx[-1], dtype=np.int32)
    out[: idx.size] = idx
    return out.reshape(n_chunks, chunk)


def _build_idx():
    u_list, m_list = [], []
    for b in range(32):
        t = np.arange(2048, dtype=np.int32) + b * 2048
        u = t[~_MASK_NP[b]]
        m = t[_MASK_NP[b]]
        u_list.append(_pad_chunks(u, 15, _CHUNK))
        m_list.append(_pad_chunks(m, 15, _ECHUNK))
    return np.stack(u_list), np.stack(m_list)


_UIDX_NP, _MIDX_NP = _build_idx()
_N_U, _N_M = _UIDX_NP.shape[1], _MIDX_NP.shape[1]


def _sc_body(seqs_hbm, embed_hbm, uidx_hbm, midx_hbm, out_hbm,
             uidx_v, midx_v, buf0, buf1, gsem, ssem0, ssem1):
    wid = lax.axis_index("s") * 2 + lax.axis_index("c")
    pltpu.sync_copy(uidx_hbm.at[wid], uidx_v)
    pltpu.sync_copy(midx_hbm.at[wid], midx_v)
    bufs = (buf0, buf1)
    ssems = (ssem0, ssem1)
    pend = [None, None]
    for j in range(_N_U):
        b = j & 1
        if pend[b] is not None:
            pend[b].wait()
        pltpu.async_copy(seqs_hbm.at[uidx_v.at[j]], bufs[b], gsem).wait()
        pend[b] = pltpu.async_copy(bufs[b], out_hbm.at[uidx_v.at[j]], ssems[b])
    for b in (0, 1):
        if pend[b] is not None:
            pend[b].wait()
    eb = buf0.at[pl.ds(0, _ECHUNK)]
    pltpu.sync_copy(embed_hbm, eb)
    epend = [pltpu.async_copy(eb, out_hbm.at[midx_v.at[j]], ssem0)
             for j in range(_N_M)]
    for c in epend:
        c.wait()


def kernel(seqs, temporal_mask_embed):
    batch, seq_len, model_dim = seqs.shape
    rows = batch * seq_len
    seqs2 = seqs.reshape(rows, model_dim)
    embed2 = jnp.broadcast_to(temporal_mask_embed[None, :], (_ECHUNK, model_dim))
    mesh = plsc.VectorSubcoreMesh(core_axis_name="c", subcore_axis_name="s")
    run = functools.partial(
        pl.kernel,
        mesh=mesh,
        out_type=jax.ShapeDtypeStruct((rows, model_dim), seqs.dtype),
        scratch_types=[
            pltpu.VMEM((_N_U, _CHUNK), jnp.int32),
            pltpu.VMEM((_N_M, _ECHUNK), jnp.int32),
            pltpu.VMEM((_CHUNK, model_dim), jnp.float32),
            pltpu.VMEM((_CHUNK, model_dim), jnp.float32),
            pltpu.SemaphoreType.DMA,
            pltpu.SemaphoreType.DMA,
            pltpu.SemaphoreType.DMA,
        ],
    )(_sc_body)
    out = run(seqs2, embed2, jnp.asarray(_UIDX_NP), jnp.asarray(_MIDX_NP))
    return (out.reshape(batch, seq_len, model_dim), jnp.asarray(_MASK_NP))


# speedup vs baseline: 1.1427x; 1.1427x over previous
import functools

import jax
import jax.numpy as jnp
import numpy as np
from jax import lax
from jax.experimental import pallas as pl
from jax.experimental.pallas import tpu as pltpu
from jax.experimental.pallas import tpu_sc as plsc


def _span_mask(key, num_rows, max_row_len, span_len, max_mask_prob):
    row_lens = jnp.full((num_rows,), max_row_len, dtype=jnp.int32)
    num_spans = int(np.float32(max_mask_prob / span_len) * np.float32(max_row_len - 1))
    k1, k2 = jax.random.split(key)
    span_start_range = row_lens - span_len + 1
    span_start_range = jnp.repeat(span_start_range, num_spans)
    rand_scales = jax.random.uniform(k1, (num_rows * num_spans,), dtype=jnp.float32)
    span_offsets = (span_start_range.astype(jnp.float32) * rand_scales).astype(jnp.int32)
    span_offsets = span_offsets.reshape(num_rows, num_spans)
    span_offsets = jnp.repeat(span_offsets, span_len, axis=1)
    idx = jnp.tile(jnp.arange(span_len, dtype=jnp.int32), num_spans)[None, :]
    indices = span_offsets + idx
    row_ids = jnp.arange(num_rows, dtype=jnp.int32)[:, None]
    float_mask = jnp.zeros((num_rows, max_row_len), dtype=jnp.float32).at[row_ids, indices].set(1.0)
    min_num_masked = jnp.count_nonzero(float_mask, axis=-1).min()
    scores = jnp.where(float_mask > 0, jax.random.uniform(k2, float_mask.shape), -1.0)
    k_max = num_spans * span_len
    _, topk_idx = jax.lax.top_k(scores, k_max)
    keep = jnp.arange(k_max) < min_num_masked
    bool_mask = jnp.zeros((num_rows, max_row_len), dtype=bool).at[row_ids, topk_idx].set(keep)
    return bool_mask


_MASK_NP = np.asarray(_span_mask(jax.random.key(42), 32, 2048, 10, 0.65))

_NW = 32
_CHUNK = 64


def _pad_chunks(idx, n_chunks, chunk):
    out = np.full((n_chunks * chunk,), idx[-1], dtype=np.int32)
    out[: idx.size] = idx
    return out.reshape(n_chunks, chunk)


def _build_idx():
    u_list, m_list = [], []
    for b in range(32):
        t = np.arange(2048, dtype=np.int32) + b * 2048
        u = t[~_MASK_NP[b]]
        m = t[_MASK_NP[b]]
        u_list.append(_pad_chunks(u, 18, _CHUNK))
        m_list.append(_pad_chunks(m, 15, _CHUNK))
    return np.stack(u_list), np.stack(m_list)


_UIDX_NP, _MIDX_NP = _build_idx()
_N_U, _N_M = _UIDX_NP.shape[1], _MIDX_NP.shape[1]


def _sc_body(seqs_hbm, embed_hbm, uidx_hbm, midx_hbm, out_hbm,
             uidx_v, midx_v, buf0, buf1, gsem0, gsem1, ssem0, ssem1):
    wid = lax.axis_index("s") * 2 + lax.axis_index("c")
    pltpu.sync_copy(uidx_hbm.at[wid], uidx_v)
    pltpu.sync_copy(midx_hbm.at[wid], midx_v)
    bufs = (buf0, buf1)
    gsems = (gsem0, gsem1)
    ssems = (ssem0, ssem1)
    gpend = [None, None]
    spend = [None, None]
    gpend[0] = pltpu.async_copy(seqs_hbm.at[uidx_v.at[0]], buf0, gsem0)
    for j in range(_N_U):
        b = j & 1
        nb = b ^ 1
        gpend[b].wait()
        if j + 1 < _N_U:
            if spend[nb] is not None:
                spend[nb].wait()
            gpend[nb] = pltpu.async_copy(
                seqs_hbm.at[uidx_v.at[j + 1]], bufs[nb], gsems[nb])
        spend[b] = pltpu.async_copy(bufs[b], out_hbm.at[uidx_v.at[j]], ssems[b])
    for b in (0, 1):
        if spend[b] is not None:
            spend[b].wait()
    pltpu.sync_copy(embed_hbm, buf0)
    epend = [pltpu.async_copy(buf0, out_hbm.at[midx_v.at[j]], ssem0)
             for j in range(_N_M)]
    for c in epend:
        c.wait()


def kernel(seqs, temporal_mask_embed):
    batch, seq_len, model_dim = seqs.shape
    rows = batch * seq_len
    seqs2 = seqs.reshape(rows, model_dim)
    embed2 = jnp.broadcast_to(temporal_mask_embed[None, :], (_CHUNK, model_dim))
    mesh = plsc.VectorSubcoreMesh(core_axis_name="c", subcore_axis_name="s")
    run = functools.partial(
        pl.kernel,
        mesh=mesh,
        out_type=jax.ShapeDtypeStruct((rows, model_dim), seqs.dtype),
        scratch_types=[
            pltpu.VMEM((_N_U, _CHUNK), jnp.int32),
            pltpu.VMEM((_N_M, _CHUNK), jnp.int32),
            pltpu.VMEM((_CHUNK, model_dim), jnp.float32),
            pltpu.VMEM((_CHUNK, model_dim), jnp.float32),
            pltpu.SemaphoreType.DMA,
            pltpu.SemaphoreType.DMA,
            pltpu.SemaphoreType.DMA,
            pltpu.SemaphoreType.DMA,
        ],
    )(_sc_body)
    out = run(seqs2, embed2, jnp.asarray(_UIDX_NP), jnp.asarray(_MIDX_NP))
    return (out.reshape(batch, seq_len, model_dim), jnp.asarray(_MASK_NP))
